# Initial kernel scaffold; baseline (speedup 1.0000x reference)
import jax
import jax.numpy as jnp
from jax.experimental import pallas as pl


def kernel(features, edge_index, W1, b1, W2, b2):
    raise NotImplementedError("write your pallas kernel here")


# speedup vs baseline: 4.7368x; 4.7368x over previous
import functools

import jax
import jax.numpy as jnp
from jax import lax
from jax.experimental import pallas as pl
from jax.experimental.pallas import tpu as pltpu
from jax.experimental.pallas import tpu_sc as plsc

N, E, D = 10000, 320000, 128
NC, NS = 2, 16
NW = NC * NS
EDGES_PER_TILE = E // NW
CHUNK = 80
NCHUNK = EDGES_PER_TILE // CHUNK
N_PAD = 10240
ROWS_PER_TILE = N_PAD // NS
ZROWS = 128


def _make_sc_aggregate():
    mesh = plsc.VectorSubcoreMesh(core_axis_name="c", subcore_axis_name="s")

    @functools.partial(
        pl.kernel,
        mesh=mesh,
        out_type=jax.ShapeDtypeStruct((NC, N_PAD, D), jnp.float32),
        scratch_types=[
            pltpu.VMEM((CHUNK,), jnp.int32),
            pltpu.VMEM((CHUNK,), jnp.int32),
            pltpu.VMEM((CHUNK, D), jnp.float32),
            pltpu.VMEM((ZROWS, D), jnp.float32),
            pltpu.VMEM_SHARED((N_PAD, D), jnp.float32),
            pltpu.SemaphoreType.DMA,
        ],
    )
    def agg(table_hbm, src_hbm, dst_hbm, out_hbm,
            src_v, dst_v, rows_v, zero_v, acc_sh, sem):
        cid = lax.axis_index("c")
        sid = lax.axis_index("s")
        wid = cid * NS + sid

        def zero_row(r, carry):
            for c in range(D // 16):
                zero_v[r, pl.ds(c * 16, 16)] = jnp.zeros((16,), jnp.float32)
            return carry
        lax.fori_loop(0, ZROWS, zero_row, 0)
        for j in range(ROWS_PER_TILE // ZROWS):
            pltpu.sync_copy(
                zero_v, acc_sh.at[pl.ds(sid * ROWS_PER_TILE + j * ZROWS, ZROWS)])
        plsc.subcore_barrier()

        base0 = wid * EDGES_PER_TILE

        def chunk_body(k, carry):
            base = base0 + k * CHUNK
            pltpu.sync_copy(src_hbm.at[pl.ds(base, CHUNK)], src_v)
            pltpu.sync_copy(dst_hbm.at[pl.ds(base, CHUNK)], dst_v)
            pltpu.async_copy(table_hbm.at[src_v], rows_v, sem).wait()
            pltpu.sync_copy(rows_v, acc_sh.at[dst_v], add=True)
            return carry
        lax.fori_loop(0, NCHUNK, chunk_body, 0)

        plsc.subcore_barrier()
        pltpu.sync_copy(
            acc_sh.at[pl.ds(sid * ROWS_PER_TILE, ROWS_PER_TILE)],
            out_hbm.at[cid, pl.ds(sid * ROWS_PER_TILE, ROWS_PER_TILE)])

    return agg


_sc_aggregate = _make_sc_aggregate()


def _linear(partials, W, b, relu):
    n, d = N, partials.shape[2]
    h = W.shape[1]
    bn = 1000

    def body(p_ref, w_ref, b_ref, o_ref):
        acc = p_ref[0] + p_ref[1]
        y = jnp.dot(acc, w_ref[...], preferred_element_type=jnp.float32)
        y = y + b_ref[...]
        if relu:
            y = jnp.maximum(y, 0.0)
        o_ref[...] = y

    return pl.pallas_call(
        body,
        grid=(n // bn,),
        in_specs=[
            pl.BlockSpec((NC, bn, d), lambda i: (0, i, 0)),
            pl.BlockSpec((d, h), lambda i: (0, 0)),
            pl.BlockSpec((1, h), lambda i: (0, 0)),
        ],
        out_specs=pl.BlockSpec((bn, h), lambda i: (i, 0)),
        out_shape=jax.ShapeDtypeStruct((n, h), jnp.float32),
    )(partials, W, b.reshape(1, h))


def kernel(features, edge_index, W1, b1, W2, b2):
    x = features[0]
    src = edge_index[0]
    dst = edge_index[1]
    p1 = _sc_aggregate(x, src, dst)
    hidden = _linear(p1, W1, b1, relu=True)
    p2 = _sc_aggregate(hidden, src, dst)
    y = _linear(p2, W2, b2, relu=False)
    return y[None]


# speedup vs baseline: 10.7817x; 2.2762x over previous
import functools

import jax
import jax.numpy as jnp
from jax import lax
from jax.experimental import pallas as pl
from jax.experimental.pallas import tpu as pltpu
from jax.experimental.pallas import tpu_sc as plsc

N, E, D = 10000, 320000, 128
NC, NS = 2, 16
NW = NC * NS
EDGES_PER_TILE = E // NW
CHUNK = 80
NCHUNK = EDGES_PER_TILE // CHUNK
N_PAD = 10240
ROWS_PER_TILE = N_PAD // NS
ZROWS = 128


def _make_sc_aggregate():
    mesh = plsc.VectorSubcoreMesh(core_axis_name="c", subcore_axis_name="s")

    @functools.partial(
        pl.kernel,
        mesh=mesh,
        out_type=jax.ShapeDtypeStruct((NC, N_PAD, D), jnp.float32),
        scratch_types=[
            pltpu.VMEM((CHUNK,), jnp.int32),
            pltpu.VMEM((CHUNK,), jnp.int32),
            pltpu.VMEM((CHUNK,), jnp.int32),
            pltpu.VMEM((CHUNK,), jnp.int32),
            pltpu.VMEM((CHUNK,), jnp.int32),
            pltpu.VMEM((CHUNK,), jnp.int32),
            pltpu.VMEM((CHUNK, D), jnp.float32),
            pltpu.VMEM((CHUNK, D), jnp.float32),
            pltpu.VMEM((ZROWS, D), jnp.float32),
            pltpu.VMEM_SHARED((N_PAD, D), jnp.float32),
            pltpu.SemaphoreType.DMA,
            pltpu.SemaphoreType.DMA,
            pltpu.SemaphoreType.DMA,
            pltpu.SemaphoreType.DMA,
            pltpu.SemaphoreType.DMA,
            pltpu.SemaphoreType.DMA,
        ],
    )
    def agg(table_hbm, src_hbm, dst_hbm, out_hbm,
            srci0, srci1, dsti0, dsti1, sdsti0, sdsti1, rows0, rows1,
            zero_v, acc_sh, isem0, isem1, gsem0, gsem1, ssem0, ssem1):
        cid = lax.axis_index("c")
        sid = lax.axis_index("s")
        wid = cid * NS + sid
        srci = (srci0, srci1)
        dsti = (dsti0, dsti1)
        sdsti = (sdsti0, sdsti1)
        bufs = (rows0, rows1)
        isem = (isem0, isem1)
        gsem = (gsem0, gsem1)
        ssem = (ssem0, ssem1)
        base0 = wid * EDGES_PER_TILE

        def zero_row(r, carry):
            for c in range(D // 16):
                zero_v[r, pl.ds(c * 16, 16)] = jnp.zeros((16,), jnp.float32)
            return carry
        lax.fori_loop(0, ZROWS, zero_row, 0)
        for j in range(ROWS_PER_TILE // ZROWS):
            pltpu.sync_copy(
                zero_v, acc_sh.at[pl.ds(sid * ROWS_PER_TILE + j * ZROWS, ZROWS)])
        plsc.subcore_barrier()

        def idx_fetch(k, b):
            pltpu.async_copy(src_hbm.at[pl.ds(base0 + k * CHUNK, CHUNK)],
                             srci[b], isem[b])
            pltpu.async_copy(dst_hbm.at[pl.ds(base0 + k * CHUNK, CHUNK)],
                             dsti[b], isem[b])

        def wait_idx(k, b):
            pltpu.make_async_copy(src_hbm.at[pl.ds(base0 + k * CHUNK, CHUNK)],
                                  srci[b], isem[b]).wait()
            pltpu.make_async_copy(dst_hbm.at[pl.ds(base0 + k * CHUNK, CHUNK)],
                                  dsti[b], isem[b]).wait()

        def gather(k, b):
            pltpu.async_copy(table_hbm.at[srci[b]], bufs[b], gsem[b])

        def wait_gather(k, b):
            pltpu.make_async_copy(table_hbm.at[srci[b]], bufs[b], gsem[b]).wait()

        def stage_dst(b):
            for i in range(CHUNK // 16):
                sdsti[b][pl.ds(i * 16, 16)] = dsti[b][pl.ds(i * 16, 16)]

        def scatter(k, b):
            pltpu.async_copy(bufs[b], acc_sh.at[sdsti[b]], ssem[b], add=True)

        def wait_scatter(k, b):
            pltpu.make_async_copy(bufs[b], acc_sh.at[sdsti[b]], ssem[b]).wait()

        idx_fetch(0, 0)
        idx_fetch(1, 1)
        wait_idx(0, 0)
        gather(0, 0)
        wait_idx(1, 1)
        gather(1, 1)
        wait_gather(0, 0)
        stage_dst(0)
        idx_fetch(2, 0)
        scatter(0, 0)

        def pair_body(g, carry):
            for b in (1, 0):
                k = 2 * g + 1 if b == 1 else 2 * g + 2
                wait_scatter(k - 1, 1 - b)
                wait_idx(k + 1, 1 - b)
                gather(k + 1, 1 - b)
                wait_gather(k, b)
                stage_dst(b)
                idx_fetch(k + 2, b)
                scatter(k, b)
            return carry
        lax.fori_loop(0, (NCHUNK - 3) // 2, pair_body, 0)

        wait_scatter(NCHUNK - 3, 0)
        wait_idx(NCHUNK - 1, 0)
        gather(NCHUNK - 1, 0)
        wait_gather(NCHUNK - 2, 1)
        stage_dst(1)
        scatter(NCHUNK - 2, 1)
        wait_scatter(NCHUNK - 2, 1)
        wait_gather(NCHUNK - 1, 0)
        stage_dst(0)
        scatter(NCHUNK - 1, 0)
        wait_scatter(NCHUNK - 1, 0)

        plsc.subcore_barrier()
        pltpu.sync_copy(
            acc_sh.at[pl.ds(sid * ROWS_PER_TILE, ROWS_PER_TILE)],
            out_hbm.at[cid, pl.ds(sid * ROWS_PER_TILE, ROWS_PER_TILE)])

    return agg


_sc_aggregate = _make_sc_aggregate()


def _linear(partials, W, b, relu):
    n, d = N, partials.shape[2]
    h = W.shape[1]
    bn = 1000

    def body(p_ref, w_ref, b_ref, o_ref):
        acc = p_ref[0] + p_ref[1]
        y = jnp.dot(acc, w_ref[...], preferred_element_type=jnp.float32)
        y = y + b_ref[...]
        if relu:
            y = jnp.maximum(y, 0.0)
        o_ref[...] = y

    return pl.pallas_call(
        body,
        grid=(n // bn,),
        in_specs=[
            pl.BlockSpec((NC, bn, d), lambda i: (0, i, 0)),
            pl.BlockSpec((d, h), lambda i: (0, 0)),
            pl.BlockSpec((1, h), lambda i: (0, 0)),
        ],
        out_specs=pl.BlockSpec((bn, h), lambda i: (i, 0)),
        out_shape=jax.ShapeDtypeStruct((n, h), jnp.float32),
    )(partials, W, b.reshape(1, h))


def kernel(features, edge_index, W1, b1, W2, b2):
    x = features[0]
    src = edge_index[0]
    dst = edge_index[1]
    p1 = _sc_aggregate(x, src, dst)
    hidden = _linear(p1, W1, b1, relu=True)
    p2 = _sc_aggregate(hidden, src, dst)
    y = _linear(p2, W2, b2, relu=False)
    return y[None]


# speedup vs baseline: 12.7095x; 1.1788x over previous
import functools

import jax
import jax.numpy as jnp
from jax import lax
from jax.experimental import pallas as pl
from jax.experimental.pallas import tpu as pltpu
from jax.experimental.pallas import tpu_sc as plsc

N, E, D = 10000, 320000, 128
NC, NS = 2, 16
NW = NC * NS
EDGES_PER_TILE = E // NW
CHUNK = 80
NCHUNK = EDGES_PER_TILE // CHUNK
N_PAD = 10240
ROWS_PER_TILE = N_PAD // NS


def _make_sc_aggregate():
    mesh = plsc.VectorSubcoreMesh(core_axis_name="c", subcore_axis_name="s")

    @functools.partial(
        pl.kernel,
        mesh=mesh,
        out_type=jax.ShapeDtypeStruct((NC, N_PAD, D), jnp.float32),
        scratch_types=(
            [pltpu.VMEM((CHUNK,), jnp.int32)] * 4
            + [pltpu.VMEM((CHUNK,), jnp.int32)] * 4
            + [pltpu.VMEM((CHUNK,), jnp.int32)] * 4
            + [pltpu.VMEM((CHUNK, D), jnp.float32)] * 4
            + [pltpu.VMEM_SHARED((N_PAD, D), jnp.float32)]
            + [pltpu.SemaphoreType.DMA] * 12
        ),
    )
    def agg(table_hbm, src_hbm, dst_hbm, out_hbm, *scratch):
        cid = lax.axis_index("c")
        sid = lax.axis_index("s")
        wid = cid * NS + sid
        srci = scratch[0:4]
        dsti = scratch[4:8]
        sdsti = scratch[8:12]
        bufs = scratch[12:16]
        acc_sh = scratch[16]
        isem = scratch[17:21]
        gsem = scratch[21:25]
        ssem = scratch[25:29]
        base0 = wid * EDGES_PER_TILE

        def zero_row(r, carry):
            for c in range(D // 16):
                bufs[0][r, pl.ds(c * 16, 16)] = jnp.zeros((16,), jnp.float32)
            return carry
        lax.fori_loop(0, CHUNK, zero_row, 0)
        for j in range(ROWS_PER_TILE // CHUNK):
            pltpu.sync_copy(
                bufs[0],
                acc_sh.at[pl.ds(sid * ROWS_PER_TILE + j * CHUNK, CHUNK)])
        plsc.subcore_barrier()

        def idx_fetch(k, b):
            pltpu.async_copy(src_hbm.at[pl.ds(base0 + k * CHUNK, CHUNK)],
                             srci[b], isem[b])
            pltpu.async_copy(dst_hbm.at[pl.ds(base0 + k * CHUNK, CHUNK)],
                             dsti[b], isem[b])

        def wait_idx(k, b):
            pltpu.make_async_copy(src_hbm.at[pl.ds(base0 + k * CHUNK, CHUNK)],
                                  srci[b], isem[b]).wait()
            pltpu.make_async_copy(dst_hbm.at[pl.ds(base0 + k * CHUNK, CHUNK)],
                                  dsti[b], isem[b]).wait()

        def gather(k, b):
            pltpu.async_copy(table_hbm.at[srci[b]], bufs[b], gsem[b])

        def wait_gather(k, b):
            pltpu.make_async_copy(table_hbm.at[srci[b]], bufs[b], gsem[b]).wait()

        def stage_dst(b):
            for i in range(CHUNK // 16):
                sdsti[b][pl.ds(i * 16, 16)] = dsti[b][pl.ds(i * 16, 16)]

        def scatter(k, b):
            pltpu.async_copy(bufs[b], acc_sh.at[sdsti[b]], ssem[b], add=True)

        def wait_scatter(k, b):
            pltpu.make_async_copy(bufs[b], acc_sh.at[sdsti[b]], ssem[b]).wait()

        def step(k, b, ws=True, g=True, f=True):
            if ws:
                wait_scatter(k - 2, (b + 2) % 4)
            if g:
                wait_idx(k + 2, (b + 2) % 4)
                gather(k + 2, (b + 2) % 4)
            wait_gather(k, b)
            stage_dst(b)
            if f:
                idx_fetch(k + 4, b)
            scatter(k, b)

        for j in range(4):
            idx_fetch(j, j)
        wait_idx(0, 0)
        gather(0, 0)
        wait_idx(1, 1)
        gather(1, 1)
        step(0, 0, ws=False)
        step(1, 1, ws=False)

        def quad_body(g, carry):
            for j in range(4):
                step(2 + 4 * g + j, (2 + j) % 4)
            return carry
        lax.fori_loop(0, (NCHUNK - 9) // 4, quad_body, 0)

        step(NCHUNK - 7, 2)
        step(NCHUNK - 6, 3)
        step(NCHUNK - 5, 0)
        step(NCHUNK - 4, 1, f=False)
        step(NCHUNK - 3, 2, f=False)
        step(NCHUNK - 2, 3, g=False, f=False)
        step(NCHUNK - 1, 0, g=False, f=False)
        wait_scatter(NCHUNK - 2, 3)
        wait_scatter(NCHUNK - 1, 0)

        plsc.subcore_barrier()
        pltpu.sync_copy(
            acc_sh.at[pl.ds(sid * ROWS_PER_TILE, ROWS_PER_TILE)],
            out_hbm.at[cid, pl.ds(sid * ROWS_PER_TILE, ROWS_PER_TILE)])

    return agg


_sc_aggregate = _make_sc_aggregate()


def _linear(partials, W, b, relu):
    n, d = N, partials.shape[2]
    h = W.shape[1]
    bn = 1000

    def body(p_ref, w_ref, b_ref, o_ref):
        acc = p_ref[0] + p_ref[1]
        y = jnp.dot(acc, w_ref[...], preferred_element_type=jnp.float32)
        y = y + b_ref[...]
        if relu:
            y = jnp.maximum(y, 0.0)
        o_ref[...] = y

    return pl.pallas_call(
        body,
        grid=(n // bn,),
        in_specs=[
            pl.BlockSpec((NC, bn, d), lambda i: (0, i, 0)),
            pl.BlockSpec((d, h), lambda i: (0, 0)),
            pl.BlockSpec((1, h), lambda i: (0, 0)),
        ],
        out_specs=pl.BlockSpec((bn, h), lambda i: (i, 0)),
        out_shape=jax.ShapeDtypeStruct((n, h), jnp.float32),
    )(partials, W, b.reshape(1, h))


def kernel(features, edge_index, W1, b1, W2, b2):
    x = features[0]
    src = edge_index[0]
    dst = edge_index[1]
    p1 = _sc_aggregate(x, src, dst)
    hidden = _linear(p1, W1, b1, relu=True)
    p2 = _sc_aggregate(hidden, src, dst)
    y = _linear(p2, W2, b2, relu=False)
    return y[None]
